# baseline (speedup 1.0000x reference)
import functools

import jax
import jax.numpy as jnp
from jax import lax
from jax.experimental import pallas as pl
from jax.experimental.pallas import tpu as pltpu
from jax.experimental.pallas import tpu_sc as plsc

_L = 2048
_D = 768
_H = 64
_E = 64
_Q = 4
_NQ = _E // _Q
_RB = 256
_NRB = _L // _RB
_TP = 64
_TF = 256
_CX = 9216
_NBF = 40
_NSEGT = 168
_NC = 2
_NS = 16
_TPW = _L // (_NC * _NS)

_BE_OFF = _L
_QIDX_OFF = _BE_OFF + _NSEGT
_ESRC_OFF = _QIDX_OFF + _NBF
_EDST_OFF = _ESRC_OFF + _NBF
_META_N = _EDST_OFF + _NBF


def _route_meta_body(x_ref, rl_ref, rs_ref, meta_ref):
    x = x_ref[...]
    var = jnp.mean(x * x, axis=1, keepdims=True)
    ri = x * lax.rsqrt(var + 1e-6)
    ri = ri * lax.rsqrt(jnp.float32(_D)) * rs_ref[...]
    logits = lax.dot_general(ri, rl_ref[...], (((1,), (0,)), ((), ())),
                             preferred_element_type=jnp.float32)
    m = jnp.max(logits, axis=1, keepdims=True)
    ids = lax.broadcasted_iota(jnp.int32, (_L, _E), 1)
    eid = jnp.min(jnp.where(logits == m, ids, _E), axis=1, keepdims=True)
    oh = (eid == ids).astype(jnp.float32)

    row = lax.broadcasted_iota(jnp.int32, (_RB, _RB), 0)
    col = lax.broadcasted_iota(jnp.int32, (_RB, _RB), 1)
    ls = (col < row).astype(jnp.float32)
    cnt = jnp.zeros((1, _E), jnp.float32)
    rank_parts = []
    for b in range(_NRB):
        ohb = oh[b * _RB:(b + 1) * _RB, :]
        cum = lax.dot_general(ls, ohb, (((1,), (0,)), ((), ())),
                              preferred_element_type=jnp.float32) + cnt
        rank_parts.append(jnp.sum(ohb * cum, axis=1, keepdims=True))
        cnt = cnt + jnp.sum(ohb, axis=0, keepdims=True)
    rank = jnp.concatenate(rank_parts, axis=0)

    pc = jnp.floor((cnt + (_TP - 1)) * (1.0 / _TP)) * _TP
    r64 = lax.broadcasted_iota(jnp.int32, (_E, _E), 0)
    c64 = lax.broadcasted_iota(jnp.int32, (_E, _E), 1)
    uq = ((r64 < c64) & (r64 // _Q == c64 // _Q)).astype(jnp.float32)
    po_in = lax.dot_general(pc, uq, (((1,), (0,)), ((), ())),
                            preferred_element_type=jnp.float32)
    e2q = (lax.broadcasted_iota(jnp.int32, (_E, _NQ), 0) // _Q
           == lax.broadcasted_iota(jnp.int32, (_E, _NQ), 1)).astype(jnp.float32)
    qsum = lax.dot_general(pc, e2q, (((1,), (0,)), ((), ())),
                           preferred_element_type=jnp.float32)
    qpc = jnp.floor((qsum + (_TF - 1)) * (1.0 / _TF)) * _TF
    r16 = lax.broadcasted_iota(jnp.int32, (_NQ, _NQ), 0)
    c16 = lax.broadcasted_iota(jnp.int32, (_NQ, _NQ), 1)
    u16 = (r16 < c16).astype(jnp.float32)
    qpo = lax.dot_general(qpc, u16, (((1,), (0,)), ((), ())),
                          preferred_element_type=jnp.float32)
    q2e = (lax.broadcasted_iota(jnp.int32, (_NQ, _E), 0)
           == lax.broadcasted_iota(jnp.int32, (_NQ, _E), 1) // _Q).astype(jnp.float32)
    qpo_e = lax.dot_general(qpo, q2e, (((1,), (0,)), ((), ())),
                            preferred_element_type=jnp.float32)
    po = qpo_e + po_in
    pend = po + pc
    tot = jnp.sum(qpc, axis=1, keepdims=True)

    pog = jnp.sum(oh * po, axis=1, keepdims=True)
    meta_ref[0:_L, :] = (pog + rank).astype(jnp.int32)

    sseg = lax.broadcasted_iota(jnp.int32, (_NSEGT, 1), 0).astype(jnp.float32) * _TP
    be_f = jnp.sum((pend <= sseg).astype(jnp.float32), axis=1, keepdims=True)
    meta_ref[_BE_OFF:_BE_OFF + _NSEGT, :] = be_f.astype(jnp.int32)

    bi = lax.broadcasted_iota(jnp.int32, (_NBF, 1), 0).astype(jnp.float32)
    sv = jnp.minimum(bi * _TF, tot - _TF)
    qend = qpo + qpc
    meta_ref[_QIDX_OFF:_QIDX_OFF + _NBF, :] = jnp.sum(
        (qend <= sv).astype(jnp.float32), axis=1, keepdims=True).astype(jnp.int32)
    ub = tot * (1.0 / _TF)
    meta_ref[_ESRC_OFF:_ESRC_OFF + _NBF, :] = jnp.minimum(bi, ub - 1.0).astype(jnp.int32)
    meta_ref[_EDST_OFF:_EDST_OFF + _NBF, :] = jnp.where(
        bi < ub, bi, jnp.float32(_NBF)).astype(jnp.int32)


def _ffn_body(meta_r, pes_r, xs_ref, ge_ref, lin_ref, ys_ref):
    b = pl.program_id(0)

    @pl.when(meta_r[_EDST_OFF + b] < _NBF)
    def _go():
        xb = xs_ref[...]
        w0 = ge_ref[0, :, 0].reshape(_TF, _D)
        w1 = ge_ref[0, :, 1].reshape(_TF, _D)
        g0 = lax.dot_general(xb, w0, (((1,), (1,)), ((), ())),
                             preferred_element_type=jnp.float32)
        g1 = lax.dot_general(xb, w1, (((1,), (1,)), ((), ())),
                             preferred_element_type=jnp.float32)
        cseg = lax.broadcasted_iota(jnp.int32, (_TF, _TF), 1) // _TP
        rs1 = lax.broadcasted_iota(jnp.int32, (_TF, 1), 0) // _TP
        q4 = _Q * meta_r[_QIDX_OFF + b]
        erow = jnp.zeros((_TF, 1), jnp.int32)
        prow = jnp.zeros((_TF, 1), jnp.float32)
        for k in range(_Q):
            ek = meta_r[_BE_OFF + _Q * b + k]
            sel = rs1 == k
            erow = jnp.where(sel, ek - q4, erow)
            prow = prow + sel.astype(jnp.float32) * pes_r[jnp.minimum(ek, _E - 1)]
        sc2d = jnp.where(erow == cseg, prow, 0.0)
        act = jax.nn.gelu(g0) * g1 * sc2d
        ys_ref[...] = lax.dot_general(act, lin_ref[0], (((1,), (0,)), ((), ())),
                                      preferred_element_type=jnp.float32)


@functools.cache
def _sc_kernels():
    mesh = plsc.VectorSubcoreMesh(core_axis_name="c", subcore_axis_name="s",
                                  num_cores=_NC, num_subcores=_NS)
    scratch = [
        pltpu.VMEM((_TPW,), jnp.int32),
        pltpu.VMEM((_TPW, _D), jnp.float32),
        pltpu.SemaphoreType.DMA,
    ]

    @functools.partial(
        pl.kernel, mesh=mesh,
        out_type=jax.ShapeDtypeStruct((_CX, _D), jnp.float32),
        scratch_types=scratch,
    )
    def sc_scatter(x_hbm, pos_hbm, xs_hbm, idx_v, rows_v, sem):
        wid = lax.axis_index("s") * _NC + lax.axis_index("c")
        base = wid * _TPW
        pltpu.sync_copy(pos_hbm.at[pl.ds(base, _TPW)], idx_v)
        pltpu.sync_copy(x_hbm.at[pl.ds(base, _TPW)], rows_v)
        pltpu.async_copy(rows_v, xs_hbm.at[idx_v], sem).wait()

    @functools.partial(
        pl.kernel, mesh=mesh,
        out_type=jax.ShapeDtypeStruct((_L, _D), jnp.float32),
        scratch_types=scratch,
    )
    def sc_gather(ys_hbm, pos_hbm, out_hbm, idx_v, rows_v, sem):
        wid = lax.axis_index("s") * _NC + lax.axis_index("c")
        base = wid * _TPW
        pltpu.sync_copy(pos_hbm.at[pl.ds(base, _TPW)], idx_v)
        pltpu.async_copy(ys_hbm.at[idx_v], rows_v, sem).wait()
        pltpu.sync_copy(rows_v, out_hbm.at[pl.ds(base, _TPW)])

    return sc_scatter, sc_gather


@jax.jit
def kernel(x, router_scale, router_logits, gating_einsum, linear, per_expert_scale):
    B, L, D = x.shape
    x2 = x.reshape(L, D)
    rs = router_scale.reshape(1, D)
    ge5 = gating_einsum.reshape(_NQ, _Q, 2, _H, D)
    lin3 = linear.reshape(_NQ, _Q * _H, D)

    meta2 = pl.pallas_call(
        _route_meta_body,
        grid=(1,),
        in_specs=[
            pl.BlockSpec((L, D), lambda i: (0, 0)),
            pl.BlockSpec((D, _E), lambda i: (0, 0)),
            pl.BlockSpec((1, D), lambda i: (0, 0)),
        ],
        out_specs=pl.BlockSpec((_META_N, 1), lambda i: (0, 0)),
        out_shape=jax.ShapeDtypeStruct((_META_N, 1), jnp.int32),
        compiler_params=pltpu.CompilerParams(
            dimension_semantics=("arbitrary",),
        ),
    )(x2, router_logits, rs)

    meta = meta2.reshape(_META_N)

    sc_scatter, sc_gather = _sc_kernels()
    xs = sc_scatter(x2, meta)

    ys = pl.pallas_call(
        _ffn_body,
        grid_spec=pltpu.PrefetchScalarGridSpec(
            num_scalar_prefetch=2,
            grid=(_NBF,),
            in_specs=[
                pl.BlockSpec((_TF, D),
                             lambda b, m_r, p_r: (m_r[_ESRC_OFF + b], 0)),
                pl.BlockSpec((1, _Q, 2, _H, D),
                             lambda b, m_r, p_r: (m_r[_QIDX_OFF + b], 0, 0, 0, 0)),
                pl.BlockSpec((1, _Q * _H, D),
                             lambda b, m_r, p_r: (m_r[_QIDX_OFF + b], 0, 0)),
            ],
            out_specs=pl.BlockSpec((_TF, D),
                                   lambda b, m_r, p_r: (m_r[_EDST_OFF + b], 0)),
        ),
        out_shape=jax.ShapeDtypeStruct(((_NBF + 1) * _TF, D), jnp.float32),
        compiler_params=pltpu.CompilerParams(
            dimension_semantics=("arbitrary",),
        ),
    )(meta, per_expert_scale, xs, ge5, lin3)

    out2 = sc_gather(ys, meta)
    return out2.reshape(B, L, D)


# speedup vs baseline: 1.0726x; 1.0726x over previous
import functools

import jax
import jax.numpy as jnp
from jax import lax
from jax.experimental import pallas as pl
from jax.experimental.pallas import tpu as pltpu
from jax.experimental.pallas import tpu_sc as plsc

_L = 2048
_D = 768
_H = 64
_E = 64
_Q = 4
_NQ = _E // _Q
_RB = 256
_NRB = _L // _RB
_TP = 64
_TF = 256
_CX = 9216
_NBF = 40
_NSEGT = 168
_NC = 2
_NS = 16
_TPW = _L // (_NC * _NS)

_BE_OFF = _L
_QIDX_OFF = _BE_OFF + _NSEGT
_ESRC_OFF = _QIDX_OFF + _NBF
_EDST_OFF = _ESRC_OFF + _NBF
_META_N = _EDST_OFF + _NBF


def _route_meta_body(x_ref, rlt_ref, rs_ref, meta_ref):
    x = x_ref[...]
    var = jnp.mean(x * x, axis=1, keepdims=True)
    ri = x * lax.rsqrt(var + 1e-6)
    ri = ri * lax.rsqrt(jnp.float32(_D)) * rs_ref[...].reshape(1, _D)
    logits = lax.dot_general(ri, rlt_ref[...], (((1,), (1,)), ((), ())),
                             preferred_element_type=jnp.float32)
    m = jnp.max(logits, axis=1, keepdims=True)
    ids = lax.broadcasted_iota(jnp.int32, (_L, _E), 1)
    eid = jnp.min(jnp.where(logits == m, ids, _E), axis=1, keepdims=True)
    oh = (eid == ids).astype(jnp.float32)

    row = lax.broadcasted_iota(jnp.int32, (_RB, _RB), 0)
    col = lax.broadcasted_iota(jnp.int32, (_RB, _RB), 1)
    ls = (col < row).astype(jnp.float32)
    cnt = jnp.zeros((1, _E), jnp.float32)
    rank_parts = []
    for b in range(_NRB):
        ohb = oh[b * _RB:(b + 1) * _RB, :]
        cum = lax.dot_general(ls, ohb, (((1,), (0,)), ((), ())),
                              preferred_element_type=jnp.float32) + cnt
        rank_parts.append(jnp.sum(ohb * cum, axis=1, keepdims=True))
        cnt = cnt + jnp.sum(ohb, axis=0, keepdims=True)
    rank = jnp.concatenate(rank_parts, axis=0)

    pc = jnp.floor((cnt + (_TP - 1)) * (1.0 / _TP)) * _TP
    r64 = lax.broadcasted_iota(jnp.int32, (_E, _E), 0)
    c64 = lax.broadcasted_iota(jnp.int32, (_E, _E), 1)
    uq = ((r64 < c64) & (r64 // _Q == c64 // _Q)).astype(jnp.float32)
    po_in = lax.dot_general(pc, uq, (((1,), (0,)), ((), ())),
                            preferred_element_type=jnp.float32)
    e2q = (lax.broadcasted_iota(jnp.int32, (_E, _NQ), 0) // _Q
           == lax.broadcasted_iota(jnp.int32, (_E, _NQ), 1)).astype(jnp.float32)
    qsum = lax.dot_general(pc, e2q, (((1,), (0,)), ((), ())),
                           preferred_element_type=jnp.float32)
    qpc = jnp.floor((qsum + (_TF - 1)) * (1.0 / _TF)) * _TF
    r16 = lax.broadcasted_iota(jnp.int32, (_NQ, _NQ), 0)
    c16 = lax.broadcasted_iota(jnp.int32, (_NQ, _NQ), 1)
    u16 = (r16 < c16).astype(jnp.float32)
    qpo = lax.dot_general(qpc, u16, (((1,), (0,)), ((), ())),
                          preferred_element_type=jnp.float32)
    q2e = (lax.broadcasted_iota(jnp.int32, (_NQ, _E), 0)
           == lax.broadcasted_iota(jnp.int32, (_NQ, _E), 1) // _Q).astype(jnp.float32)
    qpo_e = lax.dot_general(qpo, q2e, (((1,), (0,)), ((), ())),
                            preferred_element_type=jnp.float32)
    po = qpo_e + po_in
    pend = po + pc
    tot = jnp.sum(qpc, axis=1, keepdims=True)

    pog = jnp.sum(oh * po, axis=1, keepdims=True)
    meta_ref[0:_L] = (pog + rank).astype(jnp.int32).reshape(_L)

    sseg = lax.broadcasted_iota(jnp.int32, (_NSEGT, 1), 0).astype(jnp.float32) * _TP
    be_f = jnp.sum((pend <= sseg).astype(jnp.float32), axis=1, keepdims=True)
    meta_ref[_BE_OFF:_BE_OFF + _NSEGT] = be_f.astype(jnp.int32).reshape(_NSEGT)

    bi = lax.broadcasted_iota(jnp.int32, (_NBF, 1), 0).astype(jnp.float32)
    sv = jnp.minimum(bi * _TF, tot - _TF)
    qend = qpo + qpc
    meta_ref[_QIDX_OFF:_QIDX_OFF + _NBF] = jnp.sum(
        (qend <= sv).astype(jnp.float32), axis=1, keepdims=True).astype(jnp.int32).reshape(_NBF)
    ub = tot * (1.0 / _TF)
    meta_ref[_ESRC_OFF:_ESRC_OFF + _NBF] = jnp.minimum(bi, ub - 1.0).astype(jnp.int32).reshape(_NBF)
    meta_ref[_EDST_OFF:_EDST_OFF + _NBF] = jnp.where(
        bi < ub, bi, jnp.float32(_NBF)).astype(jnp.int32).reshape(_NBF)


def _ffn_body(meta_r, pes_r, xs_ref, ge_ref, lin_ref, ys_ref):
    b = pl.program_id(0)

    @pl.when(meta_r[_EDST_OFF + b] < _NBF)
    def _go():
        xb = xs_ref[...]
        w0 = ge_ref[0, :, 0].reshape(_TF, _D)
        w1 = ge_ref[0, :, 1].reshape(_TF, _D)
        g0 = lax.dot_general(xb, w0, (((1,), (1,)), ((), ())),
                             preferred_element_type=jnp.float32)
        g1 = lax.dot_general(xb, w1, (((1,), (1,)), ((), ())),
                             preferred_element_type=jnp.float32)
        cseg = lax.broadcasted_iota(jnp.int32, (_TF, _TF), 1) // _TP
        rs1 = lax.broadcasted_iota(jnp.int32, (_TF, 1), 0) // _TP
        q4 = _Q * meta_r[_QIDX_OFF + b]
        erow = jnp.zeros((_TF, 1), jnp.int32)
        prow = jnp.zeros((_TF, 1), jnp.float32)
        for k in range(_Q):
            ek = meta_r[_BE_OFF + _Q * b + k]
            sel = rs1 == k
            erow = jnp.where(sel, ek - q4, erow)
            prow = prow + sel.astype(jnp.float32) * pes_r[jnp.minimum(ek, _E - 1)]
        sc2d = jnp.where(erow == cseg, prow, 0.0)
        act = jax.nn.gelu(g0) * g1 * sc2d
        ys_ref[...] = lax.dot_general(act, lin_ref[0], (((1,), (0,)), ((), ())),
                                      preferred_element_type=jnp.float32)


@functools.cache
def _sc_kernels():
    mesh = plsc.VectorSubcoreMesh(core_axis_name="c", subcore_axis_name="s",
                                  num_cores=_NC, num_subcores=_NS)
    scratch = [
        pltpu.VMEM((_TPW,), jnp.int32),
        pltpu.VMEM((_TPW, _D), jnp.float32),
        pltpu.SemaphoreType.DMA,
    ]

    @functools.partial(
        pl.kernel, mesh=mesh,
        out_type=jax.ShapeDtypeStruct((_CX, _D), jnp.float32),
        scratch_types=scratch,
    )
    def sc_scatter(x_hbm, pos_hbm, xs_hbm, idx_v, rows_v, sem):
        wid = lax.axis_index("s") * _NC + lax.axis_index("c")
        base = wid * _TPW
        pltpu.sync_copy(pos_hbm.at[pl.ds(base, _TPW)], idx_v)
        pltpu.sync_copy(x_hbm.at[pl.ds(base, _TPW)], rows_v)
        pltpu.async_copy(rows_v, xs_hbm.at[idx_v], sem).wait()

    @functools.partial(
        pl.kernel, mesh=mesh,
        out_type=jax.ShapeDtypeStruct((_L, _D), jnp.float32),
        scratch_types=scratch,
    )
    def sc_gather(ys_hbm, pos_hbm, out_hbm, idx_v, rows_v, sem):
        wid = lax.axis_index("s") * _NC + lax.axis_index("c")
        base = wid * _TPW
        pltpu.sync_copy(pos_hbm.at[pl.ds(base, _TPW)], idx_v)
        pltpu.async_copy(ys_hbm.at[idx_v], rows_v, sem).wait()
        pltpu.sync_copy(rows_v, out_hbm.at[pl.ds(base, _TPW)])

    return sc_scatter, sc_gather


@jax.jit
def kernel(x, router_scale, router_logits, gating_einsum, linear, per_expert_scale):
    B, L, D = x.shape
    x2 = x.reshape(L, D)
    ge5 = gating_einsum.reshape(_NQ, _Q, 2, _H, D)
    lin3 = linear.reshape(_NQ, _Q * _H, D)

    meta = pl.pallas_call(
        _route_meta_body,
        grid=(1,),
        in_specs=[
            pl.BlockSpec((L, D), lambda i: (0, 0)),
            pl.BlockSpec((_E, D), lambda i: (0, 0)),
            pl.BlockSpec((D,), lambda i: (0,)),
        ],
        out_specs=pl.BlockSpec((_META_N,), lambda i: (0,)),
        out_shape=jax.ShapeDtypeStruct((_META_N,), jnp.int32),
        compiler_params=pltpu.CompilerParams(
            dimension_semantics=("arbitrary",),
        ),
    )(x2, router_logits.T, router_scale)

    sc_scatter, sc_gather = _sc_kernels()
    xs = sc_scatter(x2, meta)

    ys = pl.pallas_call(
        _ffn_body,
        grid_spec=pltpu.PrefetchScalarGridSpec(
            num_scalar_prefetch=2,
            grid=(_NBF,),
            in_specs=[
                pl.BlockSpec((_TF, D),
                             lambda b, m_r, p_r: (m_r[_ESRC_OFF + b], 0)),
                pl.BlockSpec((1, _Q, 2, _H, D),
                             lambda b, m_r, p_r: (m_r[_QIDX_OFF + b], 0, 0, 0, 0)),
                pl.BlockSpec((1, _Q * _H, D),
                             lambda b, m_r, p_r: (m_r[_QIDX_OFF + b], 0, 0)),
            ],
            out_specs=pl.BlockSpec((_TF, D),
                                   lambda b, m_r, p_r: (m_r[_EDST_OFF + b], 0)),
        ),
        out_shape=jax.ShapeDtypeStruct(((_NBF + 1) * _TF, D), jnp.float32),
        compiler_params=pltpu.CompilerParams(
            dimension_semantics=("arbitrary",),
        ),
    )(meta, per_expert_scale, xs, ge5, lin3)

    out2 = sc_gather(ys, meta)
    return out2.reshape(B, L, D)


# speedup vs baseline: 1.0784x; 1.0055x over previous
import functools

import jax
import jax.numpy as jnp
from jax import lax
from jax.experimental import pallas as pl
from jax.experimental.pallas import tpu as pltpu
from jax.experimental.pallas import tpu_sc as plsc

_L = 2048
_D = 768
_H = 64
_E = 64
_Q = 4
_NQ = _E // _Q
_RB = 256
_NRB = _L // _RB
_TP = 64
_TF = 256
_CX = 9216
_NBF = 40
_NSEGT = 168
_NC = 2
_NS = 16
_TPW = _L // (_NC * _NS)

_BE_OFF = _L
_QIDX_OFF = _BE_OFF + _NSEGT
_ESRC_OFF = _QIDX_OFF + _NBF
_EDST_OFF = _ESRC_OFF + _NBF
_META_N = _EDST_OFF + _NBF


_HL = _L // 2


def _route_meta_body(x_ref, rlt_ref, rs_ref, meta_ref, eid_s, rank_s, cnt_s):
    j = pl.program_id(0)
    x = x_ref[...]
    var = jnp.mean(x * x, axis=1, keepdims=True)
    ri = x * lax.rsqrt(var + 1e-6)
    ri = ri * lax.rsqrt(jnp.float32(_D)) * rs_ref[...].reshape(1, _D)
    logits = lax.dot_general(ri, rlt_ref[...], (((1,), (1,)), ((), ())),
                             preferred_element_type=jnp.float32)
    m = jnp.max(logits, axis=1, keepdims=True)
    ids_h = lax.broadcasted_iota(jnp.int32, (_HL, _E), 1)
    eid = jnp.min(jnp.where(logits == m, ids_h, _E), axis=1, keepdims=True)
    oh_h = (eid == ids_h).astype(jnp.float32)

    row = lax.broadcasted_iota(jnp.int32, (_RB, _RB), 0)
    col = lax.broadcasted_iota(jnp.int32, (_RB, _RB), 1)
    ls = (col < row).astype(jnp.float32)
    cnt = jnp.where(j == 0, jnp.zeros((1, _E), jnp.float32), cnt_s[...])
    rank_parts = []
    for b in range(_HL // _RB):
        ohb = oh_h[b * _RB:(b + 1) * _RB, :]
        cum = lax.dot_general(ls, ohb, (((1,), (0,)), ((), ())),
                              preferred_element_type=jnp.float32) + cnt
        rank_parts.append(jnp.sum(ohb * cum, axis=1, keepdims=True))
        cnt = cnt + jnp.sum(ohb, axis=0, keepdims=True)
    cnt_s[...] = cnt
    base = j * _HL
    eid_s[pl.ds(base, _HL), :] = eid
    rank_s[pl.ds(base, _HL), :] = jnp.concatenate(rank_parts, axis=0)

    @pl.when(j == 1)
    def _meta():
        _emit_meta(meta_ref, eid_s, rank_s, cnt_s)


def _emit_meta(meta_ref, eid_s, rank_s, cnt_s):
    cnt = cnt_s[...]
    ids = lax.broadcasted_iota(jnp.int32, (_L, _E), 1)
    oh = (eid_s[...] == ids).astype(jnp.float32)
    rank = rank_s[...]

    pc = jnp.floor((cnt + (_TP - 1)) * (1.0 / _TP)) * _TP
    r64 = lax.broadcasted_iota(jnp.int32, (_E, _E), 0)
    c64 = lax.broadcasted_iota(jnp.int32, (_E, _E), 1)
    uq = ((r64 < c64) & (r64 // _Q == c64 // _Q)).astype(jnp.float32)
    po_in = lax.dot_general(pc, uq, (((1,), (0,)), ((), ())),
                            preferred_element_type=jnp.float32)
    e2q = (lax.broadcasted_iota(jnp.int32, (_E, _NQ), 0) // _Q
           == lax.broadcasted_iota(jnp.int32, (_E, _NQ), 1)).astype(jnp.float32)
    qsum = lax.dot_general(pc, e2q, (((1,), (0,)), ((), ())),
                           preferred_element_type=jnp.float32)
    qpc = jnp.floor((qsum + (_TF - 1)) * (1.0 / _TF)) * _TF
    r16 = lax.broadcasted_iota(jnp.int32, (_NQ, _NQ), 0)
    c16 = lax.broadcasted_iota(jnp.int32, (_NQ, _NQ), 1)
    u16 = (r16 < c16).astype(jnp.float32)
    qpo = lax.dot_general(qpc, u16, (((1,), (0,)), ((), ())),
                          preferred_element_type=jnp.float32)
    q2e = (lax.broadcasted_iota(jnp.int32, (_NQ, _E), 0)
           == lax.broadcasted_iota(jnp.int32, (_NQ, _E), 1) // _Q).astype(jnp.float32)
    qpo_e = lax.dot_general(qpo, q2e, (((1,), (0,)), ((), ())),
                            preferred_element_type=jnp.float32)
    po = qpo_e + po_in
    pend = po + pc
    tot = jnp.sum(qpc, axis=1, keepdims=True)

    pog = jnp.sum(oh * po, axis=1, keepdims=True)
    meta_ref[0:_L] = (pog + rank).astype(jnp.int32).reshape(_L)

    sseg = lax.broadcasted_iota(jnp.int32, (_NSEGT, 1), 0).astype(jnp.float32) * _TP
    be_f = jnp.sum((pend <= sseg).astype(jnp.float32), axis=1, keepdims=True)
    meta_ref[_BE_OFF:_BE_OFF + _NSEGT] = be_f.astype(jnp.int32).reshape(_NSEGT)

    bi = lax.broadcasted_iota(jnp.int32, (_NBF, 1), 0).astype(jnp.float32)
    sv = jnp.minimum(bi * _TF, tot - _TF)
    qend = qpo + qpc
    meta_ref[_QIDX_OFF:_QIDX_OFF + _NBF] = jnp.sum(
        (qend <= sv).astype(jnp.float32), axis=1, keepdims=True).astype(jnp.int32).reshape(_NBF)
    ub = tot * (1.0 / _TF)
    meta_ref[_ESRC_OFF:_ESRC_OFF + _NBF] = jnp.minimum(bi, ub - 1.0).astype(jnp.int32).reshape(_NBF)
    meta_ref[_EDST_OFF:_EDST_OFF + _NBF] = jnp.where(
        bi < ub, bi, jnp.float32(_NBF)).astype(jnp.int32).reshape(_NBF)


def _ffn_body(meta_r, pes_r, xs_ref, ge_ref, lin_ref, ys_ref):
    b = pl.program_id(0)

    @pl.when(meta_r[_EDST_OFF + b] < _NBF)
    def _go():
        xb = xs_ref[...]
        w0 = ge_ref[0, :, 0].reshape(_TF, _D)
        w1 = ge_ref[0, :, 1].reshape(_TF, _D)
        g0 = lax.dot_general(xb, w0, (((1,), (1,)), ((), ())),
                             preferred_element_type=jnp.float32)
        g1 = lax.dot_general(xb, w1, (((1,), (1,)), ((), ())),
                             preferred_element_type=jnp.float32)
        cseg = lax.broadcasted_iota(jnp.int32, (_TF, _TF), 1) // _TP
        rs1 = lax.broadcasted_iota(jnp.int32, (_TF, 1), 0) // _TP
        q4 = _Q * meta_r[_QIDX_OFF + b]
        erow = jnp.zeros((_TF, 1), jnp.int32)
        prow = jnp.zeros((_TF, 1), jnp.float32)
        for k in range(_Q):
            ek = meta_r[_BE_OFF + _Q * b + k]
            sel = rs1 == k
            erow = jnp.where(sel, ek - q4, erow)
            prow = prow + sel.astype(jnp.float32) * pes_r[jnp.minimum(ek, _E - 1)]
        sc2d = jnp.where(erow == cseg, prow, 0.0)
        act = jax.nn.gelu(g0) * g1 * sc2d
        ys_ref[...] = lax.dot_general(act, lin_ref[0], (((1,), (0,)), ((), ())),
                                      preferred_element_type=jnp.float32)


@functools.cache
def _sc_kernels():
    mesh = plsc.VectorSubcoreMesh(core_axis_name="c", subcore_axis_name="s",
                                  num_cores=_NC, num_subcores=_NS)
    scratch = [
        pltpu.VMEM((_TPW,), jnp.int32),
        pltpu.VMEM((_TPW, _D), jnp.float32),
        pltpu.SemaphoreType.DMA,
    ]

    @functools.partial(
        pl.kernel, mesh=mesh,
        out_type=jax.ShapeDtypeStruct((_CX, _D), jnp.float32),
        scratch_types=scratch,
    )
    def sc_scatter(x_hbm, pos_hbm, xs_hbm, idx_v, rows_v, sem):
        wid = lax.axis_index("s") * _NC + lax.axis_index("c")
        base = wid * _TPW
        pltpu.sync_copy(pos_hbm.at[pl.ds(base, _TPW)], idx_v)
        pltpu.sync_copy(x_hbm.at[pl.ds(base, _TPW)], rows_v)
        pltpu.async_copy(rows_v, xs_hbm.at[idx_v], sem).wait()

    @functools.partial(
        pl.kernel, mesh=mesh,
        out_type=jax.ShapeDtypeStruct((_L, _D), jnp.float32),
        scratch_types=scratch,
    )
    def sc_gather(ys_hbm, pos_hbm, out_hbm, idx_v, rows_v, sem):
        wid = lax.axis_index("s") * _NC + lax.axis_index("c")
        base = wid * _TPW
        pltpu.sync_copy(pos_hbm.at[pl.ds(base, _TPW)], idx_v)
        pltpu.async_copy(ys_hbm.at[idx_v], rows_v, sem).wait()
        pltpu.sync_copy(rows_v, out_hbm.at[pl.ds(base, _TPW)])

    return sc_scatter, sc_gather


@jax.jit
def kernel(x, router_scale, router_logits, gating_einsum, linear, per_expert_scale):
    B, L, D = x.shape
    x2 = x.reshape(L, D)
    ge5 = gating_einsum.reshape(_NQ, _Q, 2, _H, D)
    lin3 = linear.reshape(_NQ, _Q * _H, D)

    meta = pl.pallas_call(
        _route_meta_body,
        grid=(2,),
        in_specs=[
            pl.BlockSpec((_HL, D), lambda i: (i, 0)),
            pl.BlockSpec((_E, D), lambda i: (0, 0)),
            pl.BlockSpec((D,), lambda i: (0,)),
        ],
        out_specs=pl.BlockSpec((_META_N,), lambda i: (0,)),
        out_shape=jax.ShapeDtypeStruct((_META_N,), jnp.int32),
        scratch_shapes=[
            pltpu.VMEM((L, 1), jnp.int32),
            pltpu.VMEM((L, 1), jnp.float32),
            pltpu.VMEM((1, _E), jnp.float32),
        ],
        compiler_params=pltpu.CompilerParams(
            dimension_semantics=("arbitrary",),
        ),
    )(x2, router_logits.T, router_scale)

    sc_scatter, sc_gather = _sc_kernels()
    xs = sc_scatter(x2, meta)

    ys = pl.pallas_call(
        _ffn_body,
        grid_spec=pltpu.PrefetchScalarGridSpec(
            num_scalar_prefetch=2,
            grid=(_NBF,),
            in_specs=[
                pl.BlockSpec((_TF, D),
                             lambda b, m_r, p_r: (m_r[_ESRC_OFF + b], 0)),
                pl.BlockSpec((1, _Q, 2, _H, D),
                             lambda b, m_r, p_r: (m_r[_QIDX_OFF + b], 0, 0, 0, 0)),
                pl.BlockSpec((1, _Q * _H, D),
                             lambda b, m_r, p_r: (m_r[_QIDX_OFF + b], 0, 0)),
            ],
            out_specs=pl.BlockSpec((_TF, D),
                                   lambda b, m_r, p_r: (m_r[_EDST_OFF + b], 0)),
        ),
        out_shape=jax.ShapeDtypeStruct(((_NBF + 1) * _TF, D), jnp.float32),
        compiler_params=pltpu.CompilerParams(
            dimension_semantics=("arbitrary",),
        ),
    )(meta, per_expert_scale, xs, ge5, lin3)

    out2 = sc_gather(ys, meta)
    return out2.reshape(B, L, D)
